# Initial kernel scaffold; baseline (speedup 1.0000x reference)
import jax
import jax.numpy as jnp
from jax.experimental import pallas as pl


def kernel(stack, stack_pointers, stack_op, hiddens, graph_fts):
    raise NotImplementedError("write your pallas kernel here")


# speedup vs baseline: 2.1338x; 2.1338x over previous
import jax
import jax.numpy as jnp
from jax.experimental import pallas as pl
from jax.experimental.pallas import tpu as pltpu

_BB = 32


def _body(stack_ref, ptr_ref, op_ref, hid_ref, out_ref, nptr_ref):
    ptr = ptr_ref[...]
    vals = jnp.max(hid_ref[...], axis=1)
    step = jax.lax.broadcasted_iota(jnp.int32, (_BB, stack_ref.shape[1], 1), 1)
    sel = step == (ptr + 1)[:, :, None]
    out_ref[...] = jnp.where(sel, vals[:, None, :], stack_ref[...])
    a0 = op_ref[:, 0:1]
    a1 = op_ref[:, 1:2]
    a2 = op_ref[:, 2:3]
    am = jnp.where(a1 > a0, 1, 0)
    am = jnp.where(a2 > jnp.maximum(a0, a1), 2, am)
    nptr_ref[...] = jnp.maximum(ptr + am - 1, 0)


def kernel(stack, stack_pointers, stack_op, hiddens, graph_fts):
    del graph_fts
    B, T1, Hs = stack.shape
    ptr2 = stack_pointers.reshape(B, 1)
    grid = (B // _BB,)
    out, nptr = pl.pallas_call(
        _body,
        grid=grid,
        in_specs=[
            pl.BlockSpec((_BB, T1, Hs), lambda i: (i, 0, 0)),
            pl.BlockSpec((_BB, 1), lambda i: (i, 0)),
            pl.BlockSpec((_BB, 3), lambda i: (i, 0)),
            pl.BlockSpec((_BB, hiddens.shape[1], Hs), lambda i: (i, 0, 0)),
        ],
        out_specs=[
            pl.BlockSpec((_BB, T1, Hs), lambda i: (i, 0, 0)),
            pl.BlockSpec((_BB, 1), lambda i: (i, 0)),
        ],
        out_shape=[
            jax.ShapeDtypeStruct((B, T1, Hs), stack.dtype),
            jax.ShapeDtypeStruct((B, 1), jnp.int32),
        ],
    )(stack, ptr2, stack_op, hiddens)
    return out, nptr.reshape(B)


# speedup vs baseline: 2.2009x; 1.0315x over previous
import jax
import jax.numpy as jnp
from jax.experimental import pallas as pl
from jax.experimental.pallas import tpu as pltpu

_BB = 64


def _body(stack_ref, ptr_ref, op_ref, hid_ref, out_ref, nptr_ref):
    ptr = ptr_ref[...]
    vals = jnp.max(hid_ref[...], axis=1)
    step = jax.lax.broadcasted_iota(jnp.int32, (_BB, stack_ref.shape[1], 1), 1)
    sel = step == (ptr + 1)[:, :, None]
    out_ref[...] = jnp.where(sel, vals[:, None, :], stack_ref[...])
    a0 = op_ref[:, 0:1]
    a1 = op_ref[:, 1:2]
    a2 = op_ref[:, 2:3]
    am = jnp.where(a1 > a0, 1, 0)
    am = jnp.where(a2 > jnp.maximum(a0, a1), 2, am)
    nptr_ref[...] = jnp.maximum(ptr + am - 1, 0)


def kernel(stack, stack_pointers, stack_op, hiddens, graph_fts):
    del graph_fts
    B, T1, Hs = stack.shape
    ptr2 = stack_pointers.reshape(B, 1)
    grid = (B // _BB,)
    out, nptr = pl.pallas_call(
        _body,
        grid=grid,
        in_specs=[
            pl.BlockSpec((_BB, T1, Hs), lambda i: (i, 0, 0)),
            pl.BlockSpec((_BB, 1), lambda i: (i, 0)),
            pl.BlockSpec((_BB, 3), lambda i: (i, 0)),
            pl.BlockSpec((_BB, hiddens.shape[1], Hs), lambda i: (i, 0, 0)),
        ],
        out_specs=[
            pl.BlockSpec((_BB, T1, Hs), lambda i: (i, 0, 0)),
            pl.BlockSpec((_BB, 1), lambda i: (i, 0)),
        ],
        out_shape=[
            jax.ShapeDtypeStruct((B, T1, Hs), stack.dtype),
            jax.ShapeDtypeStruct((B, 1), jnp.int32),
        ],
    )(stack, ptr2, stack_op, hiddens)
    return out, nptr.reshape(B)
